# baseline (speedup 1.0000x reference)
import jax
import jax.numpy as jnp
from jax.experimental import pallas as pl

N = 10000
K = 16
D_IN = 256
D_OUT = 256
BLOCK = 400


def _fused_kernel(src_ref, neigh_ref, wagg_ref, wself_ref, out_ref):
    x = neigh_ref[...]
    parts = [x[:, k * D_IN:(k + 1) * D_IN] for k in range(K)]
    while len(parts) > 1:
        parts = [parts[i] + parts[i + 1] for i in range(0, len(parts), 2)]
    mean = parts[0] * (1.0 / K)
    h = jax.lax.dot_general(
        src_ref[...], wself_ref[...], (((1,), (0,)), ((), ())),
        preferred_element_type=jnp.float32,
    )
    h += jax.lax.dot_general(
        mean, wagg_ref[...], (((1,), (0,)), ((), ())),
        preferred_element_type=jnp.float32,
    )
    out_ref[...] = jnp.maximum(h, 0.0)


def kernel(src_node_features, neighbor_node_features, W_agg, W_self):
    n = src_node_features.shape[0]
    neigh2d = neighbor_node_features.reshape(n, K * D_IN)
    grid = (n // BLOCK,)
    return pl.pallas_call(
        _fused_kernel,
        grid=grid,
        in_specs=[
            pl.BlockSpec((BLOCK, D_IN), lambda i: (i, 0)),
            pl.BlockSpec((BLOCK, K * D_IN), lambda i: (i, 0)),
            pl.BlockSpec((D_IN, D_OUT), lambda i: (0, 0)),
            pl.BlockSpec((D_IN, D_OUT), lambda i: (0, 0)),
        ],
        out_specs=pl.BlockSpec((BLOCK, D_OUT), lambda i: (i, 0)),
        out_shape=jax.ShapeDtypeStruct((n, D_OUT), jnp.float32),
    )(src_node_features, neigh2d, W_agg, W_self)


# speedup vs baseline: 1.2945x; 1.2945x over previous
import jax
import jax.numpy as jnp
from jax import lax
from jax.experimental import pallas as pl
from jax.experimental.pallas import tpu as pltpu
from jax.experimental.pallas import tpu_sc as plsc

N = 10000
K = 16
D = 256

N_TC = 4880
N_SC = 5120

TC_BLOCK = 488
TAIL_BLOCK = 80
TAIL_OFF = N_TC // TAIL_BLOCK

NC = 2
NS = 16
NW = NC * NS
PER_W = N_SC // NW
CH = 8
NCHUNK = PER_W // CH


def _tc_main_body(src_ref, neigh_ref, wagg_ref, wself_ref, out_ref):
    neigh = neigh_ref[...]
    mean = jnp.sum(neigh, axis=1) * (1.0 / K)
    h = lax.dot_general(
        src_ref[...], wself_ref[...], (((1,), (0,)), ((), ())),
        preferred_element_type=jnp.float32,
    )
    h += lax.dot_general(
        mean, wagg_ref[...], (((1,), (0,)), ((), ())),
        preferred_element_type=jnp.float32,
    )
    out_ref[...] = jnp.maximum(h, 0.0)


def _tc_tail_body(src_ref, mean_ref, wagg_ref, wself_ref, out_ref):
    h = lax.dot_general(
        src_ref[...], wself_ref[...], (((1,), (0,)), ((), ())),
        preferred_element_type=jnp.float32,
    )
    h += lax.dot_general(
        mean_ref[...], wagg_ref[...], (((1,), (0,)), ((), ())),
        preferred_element_type=jnp.float32,
    )
    out_ref[...] = jnp.maximum(h, 0.0)


def _sc_agg_body(neigh_hbm, out_hbm, buf, obuf, sem0, sem1, osem0, osem1):
    w = lax.axis_index("s") * NC + lax.axis_index("c")
    base = N_TC + w * PER_W
    obase = w * PER_W
    sems = (sem0, sem1)
    osems = (osem0, osem1)
    in_cp = [None, None]
    out_cp = [None, None]
    in_cp[0] = pltpu.async_copy(neigh_hbm.at[pl.ds(base, CH)], buf.at[0], sem0)
    for c in range(NCHUNK):
        par = c & 1
        if c + 1 < NCHUNK:
            in_cp[1 - par] = pltpu.async_copy(
                neigh_hbm.at[pl.ds(base + (c + 1) * CH, CH)],
                buf.at[1 - par], sems[1 - par])
        in_cp[par].wait()
        if out_cp[par] is not None:
            out_cp[par].wait()

        @pl.loop(0, CH)
        def _(nn, par=par):
            @pl.loop(0, D, step=32)
            def _(dc, nn=nn, par=par):
                for half in range(2):
                    d0 = dc + 16 * half
                    acc = buf[par, nn, 0, pl.ds(d0, 16)]
                    for k in range(1, K):
                        acc = acc + buf[par, nn, k, pl.ds(d0, 16)]
                    obuf[par, nn, pl.ds(d0, 16)] = acc * (1.0 / K)

        out_cp[par] = pltpu.async_copy(
            obuf.at[par], out_hbm.at[pl.ds(obase + c * CH, CH)], osems[par])
    out_cp[0].wait()
    out_cp[1].wait()


def _sc_aggregate(neigh):
    mesh = plsc.VectorSubcoreMesh(core_axis_name="c", subcore_axis_name="s")
    f = pl.kernel(
        _sc_agg_body,
        out_type=jax.ShapeDtypeStruct((N_SC, D), jnp.float32),
        mesh=mesh,
        scratch_types=[
            pltpu.VMEM((2, CH, K, D), jnp.float32),
            pltpu.VMEM((2, CH, D), jnp.float32),
            pltpu.SemaphoreType.DMA,
            pltpu.SemaphoreType.DMA,
            pltpu.SemaphoreType.DMA,
            pltpu.SemaphoreType.DMA,
        ],
    )
    return f(neigh)


def kernel(src_node_features, neighbor_node_features, W_agg, W_self):
    mean_sc = _sc_aggregate(neighbor_node_features)
    out_main = pl.pallas_call(
        _tc_main_body,
        grid=(N_TC // TC_BLOCK,),
        in_specs=[
            pl.BlockSpec((TC_BLOCK, D), lambda i: (i, 0)),
            pl.BlockSpec((TC_BLOCK, K, D), lambda i: (i, 0, 0)),
            pl.BlockSpec((D, D), lambda i: (0, 0)),
            pl.BlockSpec((D, D), lambda i: (0, 0)),
        ],
        out_specs=pl.BlockSpec((TC_BLOCK, D), lambda i: (i, 0)),
        out_shape=jax.ShapeDtypeStruct((N_TC, D), jnp.float32),
    )(src_node_features, neighbor_node_features, W_agg, W_self)
    out_tail = pl.pallas_call(
        _tc_tail_body,
        grid=(N_SC // TAIL_BLOCK,),
        in_specs=[
            pl.BlockSpec((TAIL_BLOCK, D), lambda j: (TAIL_OFF + j, 0)),
            pl.BlockSpec((TAIL_BLOCK, D), lambda j: (j, 0)),
            pl.BlockSpec((D, D), lambda j: (0, 0)),
            pl.BlockSpec((D, D), lambda j: (0, 0)),
        ],
        out_specs=pl.BlockSpec((TAIL_BLOCK, D), lambda j: (j, 0)),
        out_shape=jax.ShapeDtypeStruct((N_SC, D), jnp.float32),
    )(src_node_features, mean_sc, W_agg, W_self)
    return jnp.concatenate([out_main, out_tail], axis=0)


# speedup vs baseline: 1.3786x; 1.0650x over previous
import jax
import jax.numpy as jnp
from jax import lax
from jax.experimental import pallas as pl
from jax.experimental.pallas import tpu as pltpu
from jax.experimental.pallas import tpu_sc as plsc

N = 10000
K = 16
D = 256

N_TC = 4880
N_SC = 5120

TC_BLOCK = 488
TAIL_BLOCK = 80
TAIL_OFF = N_TC // TAIL_BLOCK

NC = 2
NS = 16
NW = NC * NS
PER_W = N_SC // NW
CH = 8
NCHUNK = PER_W // CH


def _tc_main_body(src_ref, neigh_ref, wagg_ref, wself_ref, out_ref):
    neigh = neigh_ref[...]
    mean = jnp.sum(neigh, axis=1) * (1.0 / K)
    h = lax.dot_general(
        src_ref[...], wself_ref[...], (((1,), (0,)), ((), ())),
        preferred_element_type=jnp.float32,
    )
    h += lax.dot_general(
        mean, wagg_ref[...], (((1,), (0,)), ((), ())),
        preferred_element_type=jnp.float32,
    )
    out_ref[...] = jnp.maximum(h, 0.0)


def _tc_tail_body(src_ref, mean_ref, wagg_ref, wself_ref, out_ref):
    h = lax.dot_general(
        src_ref[...], wself_ref[...], (((1,), (0,)), ((), ())),
        preferred_element_type=jnp.float32,
    )
    h += lax.dot_general(
        mean_ref[...], wagg_ref[...], (((1,), (0,)), ((), ())),
        preferred_element_type=jnp.float32,
    )
    out_ref[...] = jnp.maximum(h, 0.0)


def _sc_agg_body(neigh_hbm, out_hbm):
    def body(in_vmem, out_vmem):
        @pl.loop(0, CH)
        def _(nn):
            @pl.loop(0, D, step=32)
            def _(dc, nn=nn):
                for half in range(2):
                    d0 = dc + 16 * half
                    acc = in_vmem[nn, 0, pl.ds(d0, 16)]
                    for k in range(1, K):
                        acc = acc + in_vmem[nn, k, pl.ds(d0, 16)]
                    out_vmem[nn, pl.ds(d0, 16)] = acc * (1.0 / K)

    pltpu.emit_pipeline(
        body,
        grid=(N_SC // CH,),
        in_specs=[pl.BlockSpec((CH, K, D), lambda i: (i + N_TC // CH, 0, 0))],
        out_specs=[pl.BlockSpec((CH, D), lambda i: (i, 0))],
        core_axis_name=("c", "s"),
        dimension_semantics=(pltpu.PARALLEL,),
    )(neigh_hbm, out_hbm)


def _sc_aggregate(neigh):
    mesh = plsc.VectorSubcoreMesh(core_axis_name="c", subcore_axis_name="s")
    f = pl.kernel(
        _sc_agg_body,
        out_type=jax.ShapeDtypeStruct((N_SC, D), jnp.float32),
        mesh=mesh,
        scratch_types=[],
    )
    return f(neigh)


def kernel(src_node_features, neighbor_node_features, W_agg, W_self):
    mean_sc = _sc_aggregate(neighbor_node_features)
    out_main = pl.pallas_call(
        _tc_main_body,
        grid=(N_TC // TC_BLOCK,),
        in_specs=[
            pl.BlockSpec((TC_BLOCK, D), lambda i: (i, 0)),
            pl.BlockSpec((TC_BLOCK, K, D), lambda i: (i, 0, 0)),
            pl.BlockSpec((D, D), lambda i: (0, 0)),
            pl.BlockSpec((D, D), lambda i: (0, 0)),
        ],
        out_specs=pl.BlockSpec((TC_BLOCK, D), lambda i: (i, 0)),
        out_shape=jax.ShapeDtypeStruct((N_TC, D), jnp.float32),
    )(src_node_features, neighbor_node_features, W_agg, W_self)
    out_tail = pl.pallas_call(
        _tc_tail_body,
        grid=(N_SC // TAIL_BLOCK,),
        in_specs=[
            pl.BlockSpec((TAIL_BLOCK, D), lambda j: (TAIL_OFF + j, 0)),
            pl.BlockSpec((TAIL_BLOCK, D), lambda j: (j, 0)),
            pl.BlockSpec((D, D), lambda j: (0, 0)),
            pl.BlockSpec((D, D), lambda j: (0, 0)),
        ],
        out_specs=pl.BlockSpec((TAIL_BLOCK, D), lambda j: (j, 0)),
        out_shape=jax.ShapeDtypeStruct((N_SC, D), jnp.float32),
    )(src_node_features, mean_sc, W_agg, W_self)
    return jnp.concatenate([out_main, out_tail], axis=0)
